# initial kernel scaffold (unmeasured)
import jax
import jax.numpy as jnp
from jax import lax
from jax.experimental import pallas as pl
from jax.experimental.pallas import tpu as pltpu

N_DEV = 16
B = 2
SQ_PER = 128
SQ = N_DEV * SQ_PER
D = 512
HQ_PER = 4
DH = 64
HD_PER = HQ_PER * DH
ROWS = B * SQ
QB = 512


def _body(x_ref, wq_ref, wk_ref, wv_ref, wo_ref, cos_ref, sin_ref,
          out_ref,
          x_full, q_ref, k_ref, v_ref, ctx_ref, acc, rs_buf,
          ag_send, ag_recv, rs_send, rs_recv, credit):
    me = lax.axis_index("i")
    left = (me + N_DEV - 1) % N_DEV
    right = (me + 1) % N_DEV

    bar = pltpu.get_barrier_semaphore()
    for nbr in (left, right):
        pl.semaphore_signal(bar, inc=1, device_id=(nbr,),
                            device_id_type=pl.DeviceIdType.MESH)
    pl.semaphore_wait(bar, 2)

    x_full[:, 0] = x_ref[...].astype(jnp.bfloat16)
    for h in range(N_DEV - 1):
        rdma = pltpu.make_async_remote_copy(
            src_ref=x_full.at[:, (N_DEV - h) % N_DEV],
            dst_ref=x_full.at[:, N_DEV - 1 - h],
            send_sem=ag_send.at[h],
            recv_sem=ag_recv.at[h],
            device_id=(right,),
            device_id_type=pl.DeviceIdType.MESH,
        )
        rdma.start()
        rdma.wait()

    x2 = x_full[...].reshape(ROWS, D)

    cos1 = cos_ref[...]
    sin1 = sin_ref[...]
    cos2 = jnp.concatenate([cos1, cos1], axis=0)
    sin2 = jnp.concatenate([sin1, sin1], axis=0)
    cos_f = jnp.concatenate([cos2] * HQ_PER, axis=1)
    sin_f = jnp.concatenate([sin2] * HQ_PER, axis=1)

    lane = lax.broadcasted_iota(jnp.int32, (ROWS, HD_PER), 1)
    even = (lane % 2) == 0

    def rope(t):
        t_l = jnp.concatenate([t[:, 1:], t[:, :1]], axis=1)
        t_r = jnp.concatenate([t[:, -1:], t[:, :-1]], axis=1)
        rot = jnp.where(even, -t_l, t_r)
        return t * cos_f + rot * sin_f

    wq = wq_ref[...].astype(jnp.bfloat16)
    wk = wk_ref[...].astype(jnp.bfloat16)
    wv = wv_ref[...].astype(jnp.bfloat16)
    q_ref[...] = rope(jnp.dot(x2, wq, preferred_element_type=jnp.float32)
                      ).astype(jnp.bfloat16)
    k_ref[...] = rope(jnp.dot(x2, wk, preferred_element_type=jnp.float32)
                      ).astype(jnp.bfloat16)
    v_ref[...] = jnp.dot(x2, wv, preferred_element_type=jnp.float32
                         ).astype(jnp.bfloat16)

    for b in range(B):
        for hh in range(HQ_PER):
            ks = k_ref[b * SQ:(b + 1) * SQ, hh * DH:(hh + 1) * DH]
            vs = v_ref[b * SQ:(b + 1) * SQ, hh * DH:(hh + 1) * DH]

            def qblk(i, _, b=b, hh=hh, ks=ks, vs=vs):
                row0 = b * SQ + i * QB
                qs = q_ref[pl.ds(row0, QB), hh * DH:(hh + 1) * DH]
                s = lax.dot_general(
                    qs, ks, (((1,), (1,)), ((), ())),
                    preferred_element_type=jnp.float32) * 0.125
                m = jnp.max(s, axis=1, keepdims=True)
                w = jnp.exp(s - m)
                w = w / jnp.sum(w, axis=1, keepdims=True)
                c = jnp.dot(w.astype(jnp.bfloat16), vs,
                            preferred_element_type=jnp.float32)
                ctx_ref[pl.ds(row0, QB), hh * DH:(hh + 1) * DH] = (
                    c.astype(jnp.bfloat16))
                return 0

            lax.fori_loop(0, SQ // QB, qblk, 0)

    wo = wo_ref[...].astype(jnp.bfloat16)
    part = jnp.dot(ctx_ref[...], wo, preferred_element_type=jnp.float32)
    acc[...] = part.reshape(B, N_DEV, SQ_PER, D)

    for s in range(N_DEV - 1):
        if s > 0:
            pl.semaphore_wait(credit, 1)
        rdma = pltpu.make_async_remote_copy(
            src_ref=acc.at[:, N_DEV - 1 - s],
            dst_ref=rs_buf,
            send_sem=rs_send.at[s],
            recv_sem=rs_recv.at[s],
            device_id=(right,),
            device_id_type=pl.DeviceIdType.MESH,
        )
        rdma.start()
        rdma.wait_send()
        rdma.wait_recv()
        acc[:, N_DEV - 2 - s] = acc[:, N_DEV - 2 - s] + rs_buf[...]
        if s < N_DEV - 2:
            pl.semaphore_signal(credit, inc=1, device_id=(left,),
                                device_id_type=pl.DeviceIdType.MESH)

    out_ref[...] = acc[:, 0]


def kernel(x, Wq, Wk, Wv, Wo):
    me = lax.axis_index("i")
    pos = jnp.arange(SQ, dtype=jnp.float32)
    inv = 1.0 / (10000.0 ** (jnp.arange(0, DH, 2, dtype=jnp.float32) / DH))
    ang = pos[:, None] * inv[None, :]
    cos = jnp.repeat(jnp.cos(ang), 2, axis=1)
    sin = jnp.repeat(jnp.sin(ang), 2, axis=1)
    cos = jnp.roll(cos, -me * SQ_PER, axis=0)
    sin = jnp.roll(sin, -me * SQ_PER, axis=0)

    return pl.pallas_call(
        _body,
        out_shape=jax.ShapeDtypeStruct((B, SQ_PER, D), jnp.float32),
        in_specs=[pl.BlockSpec(memory_space=pltpu.VMEM)] * 7,
        out_specs=pl.BlockSpec(memory_space=pltpu.VMEM),
        scratch_shapes=[
            pltpu.VMEM((B, N_DEV, SQ_PER, D), jnp.bfloat16),
            pltpu.VMEM((ROWS, HD_PER), jnp.bfloat16),
            pltpu.VMEM((ROWS, HD_PER), jnp.bfloat16),
            pltpu.VMEM((ROWS, HD_PER), jnp.bfloat16),
            pltpu.VMEM((ROWS, HD_PER), jnp.bfloat16),
            pltpu.VMEM((B, N_DEV, SQ_PER, D), jnp.float32),
            pltpu.VMEM((B, SQ_PER, D), jnp.float32),
            pltpu.SemaphoreType.DMA((N_DEV - 1,)),
            pltpu.SemaphoreType.DMA((N_DEV - 1,)),
            pltpu.SemaphoreType.DMA((N_DEV - 1,)),
            pltpu.SemaphoreType.DMA((N_DEV - 1,)),
            pltpu.SemaphoreType.REGULAR,
        ],
        compiler_params=pltpu.CompilerParams(collective_id=0),
    )(x, Wq, Wk, Wv, Wo, cos, sin)


# baseline (device time: 355441 ns/iter reference)
import jax
import jax.numpy as jnp
from jax import lax
from jax.experimental import pallas as pl
from jax.experimental.pallas import tpu as pltpu

N_DEV = 16
B = 2
SQ_PER = 128
SQ = N_DEV * SQ_PER
D = 512
HQ_PER = 4
DH = 64
HD_PER = HQ_PER * DH
ROWS = B * SQ
QB = 512


def _body(x_ref, wq_ref, wk_ref, wv_ref, wo_ref, cos_ref, sin_ref,
          out_ref,
          x_full, q_ref, k_ref, v_ref, ctx_ref, acc, rs_buf,
          ag_send, ag_recv, rs_send, rs_recv, credit):
    me = lax.axis_index("i")
    left = (me + N_DEV - 1) % N_DEV
    right = (me + 1) % N_DEV

    bar = pltpu.get_barrier_semaphore()
    for nbr in (left, right):
        pl.semaphore_signal(bar, inc=1, device_id=(nbr,),
                            device_id_type=pl.DeviceIdType.MESH)
    pl.semaphore_wait(bar, 2)

    x_full[:, 0] = x_ref[...].astype(jnp.bfloat16)
    for h in range(N_DEV - 1):
        rdma = pltpu.make_async_remote_copy(
            src_ref=x_full.at[:, (N_DEV - h) % N_DEV],
            dst_ref=x_full.at[:, N_DEV - 1 - h],
            send_sem=ag_send.at[h],
            recv_sem=ag_recv.at[h],
            device_id=(right,),
            device_id_type=pl.DeviceIdType.MESH,
        )
        rdma.start()
        rdma.wait()

    x2 = x_full[...].reshape(ROWS, D)

    cos1 = cos_ref[...]
    sin1 = sin_ref[...]
    cos2 = jnp.concatenate([cos1, cos1], axis=0)
    sin2 = jnp.concatenate([sin1, sin1], axis=0)
    cos_f = jnp.concatenate([cos2] * HQ_PER, axis=1)
    sin_f = jnp.concatenate([sin2] * HQ_PER, axis=1)

    lane = lax.broadcasted_iota(jnp.int32, (ROWS, HD_PER), 1)
    even = (lane % 2) == 0

    def rope(t):
        t_l = jnp.concatenate([t[:, 1:], t[:, :1]], axis=1)
        t_r = jnp.concatenate([t[:, -1:], t[:, :-1]], axis=1)
        rot = jnp.where(even, -t_l, t_r)
        return t * cos_f + rot * sin_f

    wq = wq_ref[...].astype(jnp.bfloat16)
    wk = wk_ref[...].astype(jnp.bfloat16)
    wv = wv_ref[...].astype(jnp.bfloat16)
    q_ref[...] = rope(jnp.dot(x2, wq, preferred_element_type=jnp.float32)
                      ).astype(jnp.bfloat16)
    k_ref[...] = rope(jnp.dot(x2, wk, preferred_element_type=jnp.float32)
                      ).astype(jnp.bfloat16)
    v_ref[...] = jnp.dot(x2, wv, preferred_element_type=jnp.float32
                         ).astype(jnp.bfloat16)

    for b in range(B):
        for hh in range(HQ_PER):
            ks = k_ref[b * SQ:(b + 1) * SQ, hh * DH:(hh + 1) * DH]
            vs = v_ref[b * SQ:(b + 1) * SQ, hh * DH:(hh + 1) * DH]

            def qblk(i, _, b=b, hh=hh, ks=ks, vs=vs):
                row0 = b * SQ + i * QB
                qs = q_ref[pl.ds(row0, QB), hh * DH:(hh + 1) * DH]
                s = lax.dot_general(
                    qs, ks, (((1,), (1,)), ((), ())),
                    preferred_element_type=jnp.float32) * 0.125
                m = jnp.max(s, axis=1, keepdims=True)
                w = jnp.exp(s - m)
                w = w / jnp.sum(w, axis=1, keepdims=True)
                c = jnp.dot(w.astype(jnp.bfloat16), vs,
                            preferred_element_type=jnp.float32)
                ctx_ref[pl.ds(row0, QB), hh * DH:(hh + 1) * DH] = (
                    c.astype(jnp.bfloat16))
                return 0

            lax.fori_loop(0, SQ // QB, qblk, 0)

    wo = wo_ref[...].astype(jnp.bfloat16)
    part = jnp.dot(ctx_ref[...], wo, preferred_element_type=jnp.float32)
    acc[...] = part.reshape(B, N_DEV, SQ_PER, D)

    for s in range(N_DEV - 1):
        if s > 0:
            pl.semaphore_wait(credit, 1)
        rdma = pltpu.make_async_remote_copy(
            src_ref=acc.at[:, N_DEV - 1 - s],
            dst_ref=rs_buf,
            send_sem=rs_send.at[s],
            recv_sem=rs_recv.at[s],
            device_id=(right,),
            device_id_type=pl.DeviceIdType.MESH,
        )
        rdma.start()
        rdma.wait_send()
        rdma.wait_recv()
        acc[:, N_DEV - 2 - s] = acc[:, N_DEV - 2 - s] + rs_buf[...]
        if s < N_DEV - 2:
            pl.semaphore_signal(credit, inc=1, device_id=(left,),
                                device_id_type=pl.DeviceIdType.MESH)

    out_ref[...] = acc[:, 0]


def kernel(x, Wq, Wk, Wv, Wo):
    me = lax.axis_index("i")
    pos = jnp.arange(SQ, dtype=jnp.float32)
    inv = 1.0 / (10000.0 ** (jnp.arange(0, DH, 2, dtype=jnp.float32) / DH))
    ang = pos[:, None] * inv[None, :]
    cos = jnp.repeat(jnp.cos(ang), 2, axis=1)
    sin = jnp.repeat(jnp.sin(ang), 2, axis=1)
    cos = jnp.roll(cos, -me * SQ_PER, axis=0)
    sin = jnp.roll(sin, -me * SQ_PER, axis=0)

    return pl.pallas_call(
        _body,
        out_shape=jax.ShapeDtypeStruct((B, SQ_PER, D), jnp.float32),
        in_specs=[pl.BlockSpec(memory_space=pltpu.VMEM)] * 7,
        out_specs=pl.BlockSpec(memory_space=pltpu.VMEM),
        scratch_shapes=[
            pltpu.VMEM((B, N_DEV, SQ_PER, D), jnp.bfloat16),
            pltpu.VMEM((ROWS, HD_PER), jnp.bfloat16),
            pltpu.VMEM((ROWS, HD_PER), jnp.bfloat16),
            pltpu.VMEM((ROWS, HD_PER), jnp.bfloat16),
            pltpu.VMEM((ROWS, HD_PER), jnp.bfloat16),
            pltpu.VMEM((B, N_DEV, SQ_PER, D), jnp.float32),
            pltpu.VMEM((B, SQ_PER, D), jnp.float32),
            pltpu.SemaphoreType.DMA((N_DEV - 1,)),
            pltpu.SemaphoreType.DMA((N_DEV - 1,)),
            pltpu.SemaphoreType.DMA((N_DEV - 1,)),
            pltpu.SemaphoreType.DMA((N_DEV - 1,)),
            pltpu.SemaphoreType.REGULAR,
        ],
        compiler_params=pltpu.CompilerParams(
            collective_id=0,
            vmem_limit_bytes=100 * 1024 * 1024,
        ),
    )(x, Wq, Wk, Wv, Wo, cos, sin)


# device time: 312818 ns/iter; 1.1363x vs baseline; 1.1363x over previous
import jax
import jax.numpy as jnp
from jax import lax
from jax.experimental import pallas as pl
from jax.experimental.pallas import tpu as pltpu

N_DEV = 16
B = 2
SQ_PER = 128
SQ = N_DEV * SQ_PER
D = 512
HQ_PER = 4
DH = 64
HD_PER = HQ_PER * DH
ROWS = B * SQ
QB = 512


def _body(x_ref, wq_ref, wk_ref, wv_ref, wo_ref, cos_ref, sin_ref,
          out_ref,
          x_full, q_ref, k_ref, v_ref, ctx_ref, acc, rs_buf,
          ag_send, ag_recv, rs_send, rs_recv, credit):
    me = lax.axis_index("i")
    left = (me + N_DEV - 1) % N_DEV
    right = (me + 1) % N_DEV

    bar = pltpu.get_barrier_semaphore()
    for nbr in (left, right):
        pl.semaphore_signal(bar, inc=1, device_id=(nbr,),
                            device_id_type=pl.DeviceIdType.MESH)
    pl.semaphore_wait(bar, 2)

    x_full[:, 0] = x_ref[...].astype(jnp.bfloat16)
    for h in range(N_DEV - 1):
        rdma = pltpu.make_async_remote_copy(
            src_ref=x_full.at[:, (N_DEV - h) % N_DEV],
            dst_ref=x_full.at[:, N_DEV - 1 - h],
            send_sem=ag_send.at[h],
            recv_sem=ag_recv.at[h],
            device_id=(right,),
            device_id_type=pl.DeviceIdType.MESH,
        )
        rdma.start()
        rdma.wait()

    x2 = x_full[...].reshape(ROWS, D)

    cos1 = cos_ref[...]
    sin1 = sin_ref[...]
    cos2 = jnp.concatenate([cos1, cos1], axis=0)
    sin2 = jnp.concatenate([sin1, sin1], axis=0)
    cos_f = jnp.concatenate([cos2] * HQ_PER, axis=1)
    sin_f = jnp.concatenate([sin2] * HQ_PER, axis=1)

    lane = lax.broadcasted_iota(jnp.int32, (ROWS, HD_PER), 1)
    even = (lane % 2) == 0

    def rope(t):
        t_l = jnp.concatenate([t[:, 1:], t[:, :1]], axis=1)
        t_r = jnp.concatenate([t[:, -1:], t[:, :-1]], axis=1)
        rot = jnp.where(even, -t_l, t_r)
        return t * cos_f + rot * sin_f

    wq = wq_ref[...].astype(jnp.bfloat16)
    wk = wk_ref[...].astype(jnp.bfloat16)
    wv = wv_ref[...].astype(jnp.bfloat16)
    q_ref[...] = rope(jnp.dot(x2, wq, preferred_element_type=jnp.float32)
                      ).astype(jnp.bfloat16)
    k_ref[...] = rope(jnp.dot(x2, wk, preferred_element_type=jnp.float32)
                      ).astype(jnp.bfloat16)
    v_ref[...] = jnp.dot(x2, wv, preferred_element_type=jnp.float32
                         ).astype(jnp.bfloat16)

    for b in range(B):
        for hh in range(HQ_PER):
            ks = k_ref[b * SQ:(b + 1) * SQ, hh * DH:(hh + 1) * DH]
            vs = v_ref[b * SQ:(b + 1) * SQ, hh * DH:(hh + 1) * DH]

            def qblk(i, _, b=b, hh=hh, ks=ks, vs=vs):
                row0 = b * SQ + i * QB
                qs = q_ref[pl.ds(row0, QB), hh * DH:(hh + 1) * DH]
                s = lax.dot_general(
                    qs, ks, (((1,), (1,)), ((), ())),
                    preferred_element_type=jnp.float32) * 0.125
                m = jnp.max(s, axis=1, keepdims=True)
                w = jnp.exp(s - m)
                w = w / jnp.sum(w, axis=1, keepdims=True)
                c = jnp.dot(w.astype(jnp.bfloat16), vs,
                            preferred_element_type=jnp.float32)
                ctx_ref[pl.ds(row0, QB), hh * DH:(hh + 1) * DH] = (
                    c.astype(jnp.bfloat16))
                return 0

            lax.fori_loop(0, SQ // QB, qblk, 0)

    wo = wo_ref[...].astype(jnp.bfloat16)
    part = jnp.dot(ctx_ref[...], wo, preferred_element_type=jnp.float32)
    acc[...] = part.astype(jnp.bfloat16).reshape(B, N_DEV, SQ_PER, D)

    for s in range(N_DEV - 1):
        if s > 0:
            pl.semaphore_wait(credit, 1)
        rdma = pltpu.make_async_remote_copy(
            src_ref=acc.at[:, N_DEV - 1 - s],
            dst_ref=rs_buf,
            send_sem=rs_send.at[s],
            recv_sem=rs_recv.at[s],
            device_id=(right,),
            device_id_type=pl.DeviceIdType.MESH,
        )
        rdma.start()
        rdma.wait_send()
        rdma.wait_recv()
        acc[:, N_DEV - 2 - s] = acc[:, N_DEV - 2 - s] + rs_buf[...]
        if s < N_DEV - 2:
            pl.semaphore_signal(credit, inc=1, device_id=(left,),
                                device_id_type=pl.DeviceIdType.MESH)

    out_ref[...] = acc[:, 0].astype(jnp.float32)


def kernel(x, Wq, Wk, Wv, Wo):
    me = lax.axis_index("i")
    pos = jnp.arange(SQ, dtype=jnp.float32)
    inv = 1.0 / (10000.0 ** (jnp.arange(0, DH, 2, dtype=jnp.float32) / DH))
    ang = pos[:, None] * inv[None, :]
    cos = jnp.repeat(jnp.cos(ang), 2, axis=1)
    sin = jnp.repeat(jnp.sin(ang), 2, axis=1)
    cos = jnp.roll(cos, -me * SQ_PER, axis=0)
    sin = jnp.roll(sin, -me * SQ_PER, axis=0)

    return pl.pallas_call(
        _body,
        out_shape=jax.ShapeDtypeStruct((B, SQ_PER, D), jnp.float32),
        in_specs=[pl.BlockSpec(memory_space=pltpu.VMEM)] * 7,
        out_specs=pl.BlockSpec(memory_space=pltpu.VMEM),
        scratch_shapes=[
            pltpu.VMEM((B, N_DEV, SQ_PER, D), jnp.bfloat16),
            pltpu.VMEM((ROWS, HD_PER), jnp.bfloat16),
            pltpu.VMEM((ROWS, HD_PER), jnp.bfloat16),
            pltpu.VMEM((ROWS, HD_PER), jnp.bfloat16),
            pltpu.VMEM((ROWS, HD_PER), jnp.bfloat16),
            pltpu.VMEM((B, N_DEV, SQ_PER, D), jnp.bfloat16),
            pltpu.VMEM((B, SQ_PER, D), jnp.bfloat16),
            pltpu.SemaphoreType.DMA((N_DEV - 1,)),
            pltpu.SemaphoreType.DMA((N_DEV - 1,)),
            pltpu.SemaphoreType.DMA((N_DEV - 1,)),
            pltpu.SemaphoreType.DMA((N_DEV - 1,)),
            pltpu.SemaphoreType.REGULAR,
        ],
        compiler_params=pltpu.CompilerParams(
            collective_id=0,
            vmem_limit_bytes=100 * 1024 * 1024,
        ),
    )(x, Wq, Wk, Wv, Wo, cos, sin)


# device time: 214089 ns/iter; 1.6602x vs baseline; 1.4612x over previous
import jax
import jax.numpy as jnp
from jax import lax
from jax.experimental import pallas as pl
from jax.experimental.pallas import tpu as pltpu

N_DEV = 16
B = 2
SQ_PER = 128
SQ = N_DEV * SQ_PER
D = 512
HQ_PER = 4
DH = 64
HD_PER = HQ_PER * DH
ROWS = B * SQ
QB = 512


def _body(x_ref, wq_ref, wk_ref, wv_ref, wo_ref, cos_ref, sin_ref,
          out_ref,
          x_full, q_ref, k_ref, v_ref, acc, rs_buf,
          ag_send, ag_recv, rs_send, rs_recv, credit):
    me = lax.axis_index("i")
    left = (me + N_DEV - 1) % N_DEV
    right = (me + 1) % N_DEV

    bar = pltpu.get_barrier_semaphore()
    for nbr in (left, right):
        pl.semaphore_signal(bar, inc=1, device_id=(nbr,),
                            device_id_type=pl.DeviceIdType.MESH)
    pl.semaphore_wait(bar, 2)

    x_full[:, 0] = x_ref[...].astype(jnp.bfloat16)
    for h in range(N_DEV - 1):
        rdma = pltpu.make_async_remote_copy(
            src_ref=x_full.at[:, (N_DEV - h) % N_DEV],
            dst_ref=x_full.at[:, N_DEV - 1 - h],
            send_sem=ag_send.at[h],
            recv_sem=ag_recv.at[h],
            device_id=(right,),
            device_id_type=pl.DeviceIdType.MESH,
        )
        rdma.start()
        rdma.wait()

    x2 = x_full[...].reshape(ROWS, D)

    cos1 = cos_ref[...]
    sin1 = sin_ref[...]
    cos2 = jnp.concatenate([cos1, cos1], axis=0)
    sin2 = jnp.concatenate([sin1, sin1], axis=0)
    cos_f = jnp.concatenate([cos2] * HQ_PER, axis=1)
    sin_f = jnp.concatenate([sin2] * HQ_PER, axis=1)

    lane = lax.broadcasted_iota(jnp.int32, (ROWS, HD_PER), 1)
    even = (lane % 2) == 0

    def rope(t):
        t_l = jnp.concatenate([t[:, 1:], t[:, :1]], axis=1)
        t_r = jnp.concatenate([t[:, -1:], t[:, :-1]], axis=1)
        rot = jnp.where(even, -t_l, t_r)
        return t * cos_f + rot * sin_f

    wq = wq_ref[...].astype(jnp.bfloat16)
    wk = wk_ref[...].astype(jnp.bfloat16)
    wv = wv_ref[...].astype(jnp.bfloat16)
    q_ref[...] = rope(jnp.dot(x2, wq, preferred_element_type=jnp.float32)
                      ).astype(jnp.bfloat16)
    k_ref[...] = rope(jnp.dot(x2, wk, preferred_element_type=jnp.float32)
                      ).astype(jnp.bfloat16)
    v_ref[...] = jnp.dot(x2, wv, preferred_element_type=jnp.float32
                         ).astype(jnp.bfloat16)

    wo = wo_ref[...].astype(jnp.bfloat16)

    def compute_chunk(k):
        outs = []
        for b in range(B):
            row0 = b * SQ + k * SQ_PER
            heads = []
            for hh in range(HQ_PER):
                qs = q_ref[row0:row0 + SQ_PER, hh * DH:(hh + 1) * DH]
                ks = k_ref[b * SQ:(b + 1) * SQ, hh * DH:(hh + 1) * DH]
                vs = v_ref[b * SQ:(b + 1) * SQ, hh * DH:(hh + 1) * DH]
                s = lax.dot_general(
                    qs, ks, (((1,), (1,)), ((), ())),
                    preferred_element_type=jnp.float32) * 0.125
                m = jnp.max(s, axis=1, keepdims=True)
                w = jnp.exp(s - m)
                w = w / jnp.sum(w, axis=1, keepdims=True)
                heads.append(jnp.dot(w.astype(jnp.bfloat16), vs,
                                     preferred_element_type=jnp.float32))
            outs.append(jnp.concatenate(heads, axis=1))
        ctx_k = jnp.concatenate(outs, axis=0).astype(jnp.bfloat16)
        p = jnp.dot(ctx_k, wo, preferred_element_type=jnp.float32)
        acc[:, k] = p.astype(jnp.bfloat16).reshape(B, SQ_PER, D)

    compute_chunk(N_DEV - 1)
    for s in range(N_DEV - 1):
        if s > 0:
            pl.semaphore_wait(credit, 1)
        rdma = pltpu.make_async_remote_copy(
            src_ref=acc.at[:, N_DEV - 1 - s],
            dst_ref=rs_buf,
            send_sem=rs_send.at[s],
            recv_sem=rs_recv.at[s],
            device_id=(right,),
            device_id_type=pl.DeviceIdType.MESH,
        )
        rdma.start()
        compute_chunk(N_DEV - 2 - s)
        rdma.wait_send()
        rdma.wait_recv()
        acc[:, N_DEV - 2 - s] = acc[:, N_DEV - 2 - s] + rs_buf[...]
        if s < N_DEV - 2:
            pl.semaphore_signal(credit, inc=1, device_id=(left,),
                                device_id_type=pl.DeviceIdType.MESH)

    out_ref[...] = acc[:, 0].astype(jnp.float32)


def kernel(x, Wq, Wk, Wv, Wo):
    me = lax.axis_index("i")
    pos = jnp.arange(SQ, dtype=jnp.float32)
    inv = 1.0 / (10000.0 ** (jnp.arange(0, DH, 2, dtype=jnp.float32) / DH))
    ang = pos[:, None] * inv[None, :]
    cos = jnp.repeat(jnp.cos(ang), 2, axis=1)
    sin = jnp.repeat(jnp.sin(ang), 2, axis=1)
    cos = jnp.roll(cos, -me * SQ_PER, axis=0)
    sin = jnp.roll(sin, -me * SQ_PER, axis=0)

    return pl.pallas_call(
        _body,
        out_shape=jax.ShapeDtypeStruct((B, SQ_PER, D), jnp.float32),
        in_specs=[pl.BlockSpec(memory_space=pltpu.VMEM)] * 7,
        out_specs=pl.BlockSpec(memory_space=pltpu.VMEM),
        scratch_shapes=[
            pltpu.VMEM((B, N_DEV, SQ_PER, D), jnp.bfloat16),
            pltpu.VMEM((ROWS, HD_PER), jnp.bfloat16),
            pltpu.VMEM((ROWS, HD_PER), jnp.bfloat16),
            pltpu.VMEM((ROWS, HD_PER), jnp.bfloat16),
            pltpu.VMEM((B, N_DEV, SQ_PER, D), jnp.bfloat16),
            pltpu.VMEM((B, SQ_PER, D), jnp.bfloat16),
            pltpu.SemaphoreType.DMA((N_DEV - 1,)),
            pltpu.SemaphoreType.DMA((N_DEV - 1,)),
            pltpu.SemaphoreType.DMA((N_DEV - 1,)),
            pltpu.SemaphoreType.DMA((N_DEV - 1,)),
            pltpu.SemaphoreType.REGULAR,
        ],
        compiler_params=pltpu.CompilerParams(
            collective_id=0,
            vmem_limit_bytes=100 * 1024 * 1024,
        ),
    )(x, Wq, Wk, Wv, Wo, cos, sin)


# device time: 189690 ns/iter; 1.8738x vs baseline; 1.1286x over previous
import jax
import jax.numpy as jnp
from jax import lax
from jax.experimental import pallas as pl
from jax.experimental.pallas import tpu as pltpu

N_DEV = 16
B = 2
SQ_PER = 128
SQ = N_DEV * SQ_PER
D = 512
HQ_PER = 4
DH = 64
HD_PER = HQ_PER * DH
ROWS = B * SQ
QB = 512


def _body(x_ref, wq_ref, wk_ref, wv_ref, wo_ref, cos_ref, sin_ref,
          out_ref,
          x_full, q_ref, k_ref, v_ref, acc, rs_buf,
          ag_send, ag_recv, rs_send, rs_recv, credit):
    me = lax.axis_index("i")
    left = (me + N_DEV - 1) % N_DEV
    right = (me + 1) % N_DEV

    bar = pltpu.get_barrier_semaphore()
    for nbr in (left, right):
        pl.semaphore_signal(bar, inc=1, device_id=(nbr,),
                            device_id_type=pl.DeviceIdType.MESH)
    pl.semaphore_wait(bar, 2)

    x_full[:, 0] = x_ref[...].astype(jnp.bfloat16)
    for h in range(N_DEV // 2):
        r_rdma = pltpu.make_async_remote_copy(
            src_ref=x_full.at[:, (N_DEV - h) % N_DEV],
            dst_ref=x_full.at[:, N_DEV - 1 - h],
            send_sem=ag_send.at[h],
            recv_sem=ag_recv.at[h],
            device_id=(right,),
            device_id_type=pl.DeviceIdType.MESH,
        )
        r_rdma.start()
        if h < N_DEV // 2 - 1:
            l_rdma = pltpu.make_async_remote_copy(
                src_ref=x_full.at[:, h],
                dst_ref=x_full.at[:, h + 1],
                send_sem=ag_send.at[N_DEV // 2 + h],
                recv_sem=ag_recv.at[N_DEV // 2 + h],
                device_id=(left,),
                device_id_type=pl.DeviceIdType.MESH,
            )
            l_rdma.start()
            r_rdma.wait()
            l_rdma.wait()
        else:
            r_rdma.wait()

    x2 = x_full[...].reshape(ROWS, D)

    cos1 = cos_ref[...]
    sin1 = sin_ref[...]
    cos2 = jnp.concatenate([cos1, cos1], axis=0)
    sin2 = jnp.concatenate([sin1, sin1], axis=0)
    cos_f = jnp.concatenate([cos2] * HQ_PER, axis=1)
    sin_f = jnp.concatenate([sin2] * HQ_PER, axis=1)

    lane = lax.broadcasted_iota(jnp.int32, (ROWS, HD_PER), 1)
    even = (lane % 2) == 0

    def rope(t):
        t_l = jnp.concatenate([t[:, 1:], t[:, :1]], axis=1)
        t_r = jnp.concatenate([t[:, -1:], t[:, :-1]], axis=1)
        rot = jnp.where(even, -t_l, t_r)
        return t * cos_f + rot * sin_f

    wq = wq_ref[...].astype(jnp.bfloat16)
    wk = wk_ref[...].astype(jnp.bfloat16)
    wv = wv_ref[...].astype(jnp.bfloat16)
    q_ref[...] = rope(jnp.dot(x2, wq, preferred_element_type=jnp.float32)
                      ).astype(jnp.bfloat16)
    k_ref[...] = rope(jnp.dot(x2, wk, preferred_element_type=jnp.float32)
                      ).astype(jnp.bfloat16)
    v_ref[...] = jnp.dot(x2, wv, preferred_element_type=jnp.float32
                         ).astype(jnp.bfloat16)

    wo = wo_ref[...].astype(jnp.bfloat16)

    def compute_chunk(k):
        outs = []
        for b in range(B):
            row0 = b * SQ + k * SQ_PER
            heads = []
            for hh in range(HQ_PER):
                qs = q_ref[row0:row0 + SQ_PER, hh * DH:(hh + 1) * DH]
                ks = k_ref[b * SQ:(b + 1) * SQ, hh * DH:(hh + 1) * DH]
                vs = v_ref[b * SQ:(b + 1) * SQ, hh * DH:(hh + 1) * DH]
                s = lax.dot_general(
                    qs, ks, (((1,), (1,)), ((), ())),
                    preferred_element_type=jnp.float32) * 0.125
                m = jnp.max(s, axis=1, keepdims=True)
                w = jnp.exp(s - m)
                w = w / jnp.sum(w, axis=1, keepdims=True)
                heads.append(jnp.dot(w.astype(jnp.bfloat16), vs,
                                     preferred_element_type=jnp.float32))
            outs.append(jnp.concatenate(heads, axis=1))
        ctx_k = jnp.concatenate(outs, axis=0).astype(jnp.bfloat16)
        p = jnp.dot(ctx_k, wo, preferred_element_type=jnp.float32)
        acc[:, k] = p.astype(jnp.bfloat16).reshape(B, SQ_PER, D)

    compute_chunk(N_DEV - 1)
    for s in range(N_DEV - 1):
        if s > 0:
            pl.semaphore_wait(credit, 1)
        rdma = pltpu.make_async_remote_copy(
            src_ref=acc.at[:, N_DEV - 1 - s],
            dst_ref=rs_buf,
            send_sem=rs_send.at[s],
            recv_sem=rs_recv.at[s],
            device_id=(right,),
            device_id_type=pl.DeviceIdType.MESH,
        )
        rdma.start()
        compute_chunk(N_DEV - 2 - s)
        rdma.wait_send()
        rdma.wait_recv()
        acc[:, N_DEV - 2 - s] = acc[:, N_DEV - 2 - s] + rs_buf[...]
        if s < N_DEV - 2:
            pl.semaphore_signal(credit, inc=1, device_id=(left,),
                                device_id_type=pl.DeviceIdType.MESH)

    out_ref[...] = acc[:, 0].astype(jnp.float32)


def kernel(x, Wq, Wk, Wv, Wo):
    me = lax.axis_index("i")
    pos = jnp.arange(SQ, dtype=jnp.float32)
    inv = 1.0 / (10000.0 ** (jnp.arange(0, DH, 2, dtype=jnp.float32) / DH))
    ang = pos[:, None] * inv[None, :]
    cos = jnp.repeat(jnp.cos(ang), 2, axis=1)
    sin = jnp.repeat(jnp.sin(ang), 2, axis=1)
    cos = jnp.roll(cos, -me * SQ_PER, axis=0)
    sin = jnp.roll(sin, -me * SQ_PER, axis=0)

    return pl.pallas_call(
        _body,
        out_shape=jax.ShapeDtypeStruct((B, SQ_PER, D), jnp.float32),
        in_specs=[pl.BlockSpec(memory_space=pltpu.VMEM)] * 7,
        out_specs=pl.BlockSpec(memory_space=pltpu.VMEM),
        scratch_shapes=[
            pltpu.VMEM((B, N_DEV, SQ_PER, D), jnp.bfloat16),
            pltpu.VMEM((ROWS, HD_PER), jnp.bfloat16),
            pltpu.VMEM((ROWS, HD_PER), jnp.bfloat16),
            pltpu.VMEM((ROWS, HD_PER), jnp.bfloat16),
            pltpu.VMEM((B, N_DEV, SQ_PER, D), jnp.bfloat16),
            pltpu.VMEM((B, SQ_PER, D), jnp.bfloat16),
            pltpu.SemaphoreType.DMA((N_DEV - 1,)),
            pltpu.SemaphoreType.DMA((N_DEV - 1,)),
            pltpu.SemaphoreType.DMA((N_DEV - 1,)),
            pltpu.SemaphoreType.DMA((N_DEV - 1,)),
            pltpu.SemaphoreType.REGULAR,
        ],
        compiler_params=pltpu.CompilerParams(
            collective_id=0,
            vmem_limit_bytes=100 * 1024 * 1024,
        ),
    )(x, Wq, Wk, Wv, Wo, cos, sin)


# device time: 185819 ns/iter; 1.9128x vs baseline; 1.0208x over previous
import jax
import jax.numpy as jnp
from jax import lax
from jax.experimental import pallas as pl
from jax.experimental.pallas import tpu as pltpu

N_DEV = 16
B = 2
SQ_PER = 128
SQ = N_DEV * SQ_PER
D = 512
HQ_PER = 4
DH = 64
HD_PER = HQ_PER * DH
ROWS = B * SQ
QB = 512


def _body(x_ref, wq_ref, wk_ref, wv_ref, wo_ref, cos_ref, sin_ref,
          out_ref,
          x_full, q_ref, k_ref, v_ref, acc, rs_buf,
          ag_send, ag_recv, rs_send, rs_recv, credit):
    me = lax.axis_index("i")
    left = (me + N_DEV - 1) % N_DEV
    right = (me + 1) % N_DEV

    bar = pltpu.get_barrier_semaphore()
    for nbr in (left, right):
        pl.semaphore_signal(bar, inc=1, device_id=(nbr,),
                            device_id_type=pl.DeviceIdType.MESH)
    pl.semaphore_wait(bar, 2)

    x_full[:, 0] = x_ref[...].astype(jnp.bfloat16)
    for h in range(N_DEV // 2):
        r_rdma = pltpu.make_async_remote_copy(
            src_ref=x_full.at[:, (N_DEV - h) % N_DEV],
            dst_ref=x_full.at[:, N_DEV - 1 - h],
            send_sem=ag_send.at[h],
            recv_sem=ag_recv.at[h],
            device_id=(right,),
            device_id_type=pl.DeviceIdType.MESH,
        )
        r_rdma.start()
        if h < N_DEV // 2 - 1:
            l_rdma = pltpu.make_async_remote_copy(
                src_ref=x_full.at[:, h],
                dst_ref=x_full.at[:, h + 1],
                send_sem=ag_send.at[N_DEV // 2 + h],
                recv_sem=ag_recv.at[N_DEV // 2 + h],
                device_id=(left,),
                device_id_type=pl.DeviceIdType.MESH,
            )
            l_rdma.start()
            r_rdma.wait()
            l_rdma.wait()
        else:
            r_rdma.wait()

    x2 = x_full[...].reshape(ROWS, D)

    cos1 = cos_ref[...]
    sin1 = sin_ref[...]
    cos2 = jnp.concatenate([cos1, cos1], axis=0)
    sin2 = jnp.concatenate([sin1, sin1], axis=0)
    cos_f = jnp.concatenate([cos2] * HQ_PER, axis=1)
    sin_f = jnp.concatenate([sin2] * HQ_PER, axis=1)

    lane = lax.broadcasted_iota(jnp.int32, (ROWS, HD_PER), 1)
    even = (lane % 2) == 0

    def rope(t):
        t_l = jnp.concatenate([t[:, 1:], t[:, :1]], axis=1)
        t_r = jnp.concatenate([t[:, -1:], t[:, :-1]], axis=1)
        rot = jnp.where(even, -t_l, t_r)
        return t * cos_f + rot * sin_f

    wq = wq_ref[...].astype(jnp.bfloat16)
    wk = wk_ref[...].astype(jnp.bfloat16)
    wv = wv_ref[...].astype(jnp.bfloat16)
    q_ref[...] = rope(jnp.dot(x2, wq, preferred_element_type=jnp.float32)
                      ).astype(jnp.bfloat16)
    k_ref[...] = rope(jnp.dot(x2, wk, preferred_element_type=jnp.float32)
                      ).astype(jnp.bfloat16)
    v_ref[...] = jnp.dot(x2, wv, preferred_element_type=jnp.float32
                         ).astype(jnp.bfloat16)

    wo = wo_ref[...].astype(jnp.bfloat16)

    def compute_chunk(k):
        outs = []
        for b in range(B):
            row0 = b * SQ + k * SQ_PER
            heads = []
            for hh in range(HQ_PER):
                qs = q_ref[row0:row0 + SQ_PER, hh * DH:(hh + 1) * DH]
                ks = k_ref[b * SQ:(b + 1) * SQ, hh * DH:(hh + 1) * DH]
                vs = v_ref[b * SQ:(b + 1) * SQ, hh * DH:(hh + 1) * DH]
                s = lax.dot_general(
                    qs, ks, (((1,), (1,)), ((), ())),
                    preferred_element_type=jnp.float32) * 0.125
                w = jnp.exp(s)
                r = 1.0 / jnp.sum(w, axis=1, keepdims=True)
                heads.append(jnp.dot(w.astype(jnp.bfloat16), vs,
                                     preferred_element_type=jnp.float32) * r)
            outs.append(jnp.concatenate(heads, axis=1))
        ctx_k = jnp.concatenate(outs, axis=0).astype(jnp.bfloat16)
        p = jnp.dot(ctx_k, wo, preferred_element_type=jnp.float32)
        acc[:, k] = p.astype(jnp.bfloat16).reshape(B, SQ_PER, D)

    compute_chunk(N_DEV - 1)
    for s in range(N_DEV - 1):
        if s > 0:
            pl.semaphore_wait(credit, 1)
        rdma = pltpu.make_async_remote_copy(
            src_ref=acc.at[:, N_DEV - 1 - s],
            dst_ref=rs_buf,
            send_sem=rs_send.at[s],
            recv_sem=rs_recv.at[s],
            device_id=(right,),
            device_id_type=pl.DeviceIdType.MESH,
        )
        rdma.start()
        compute_chunk(N_DEV - 2 - s)
        rdma.wait_send()
        rdma.wait_recv()
        acc[:, N_DEV - 2 - s] = acc[:, N_DEV - 2 - s] + rs_buf[...]
        if s < N_DEV - 2:
            pl.semaphore_signal(credit, inc=1, device_id=(left,),
                                device_id_type=pl.DeviceIdType.MESH)

    out_ref[...] = acc[:, 0].astype(jnp.float32)


def kernel(x, Wq, Wk, Wv, Wo):
    me = lax.axis_index("i")
    pos = jnp.arange(SQ, dtype=jnp.float32)
    inv = 1.0 / (10000.0 ** (jnp.arange(0, DH, 2, dtype=jnp.float32) / DH))
    ang = pos[:, None] * inv[None, :]
    cos = jnp.repeat(jnp.cos(ang), 2, axis=1)
    sin = jnp.repeat(jnp.sin(ang), 2, axis=1)
    cos = jnp.roll(cos, -me * SQ_PER, axis=0)
    sin = jnp.roll(sin, -me * SQ_PER, axis=0)

    return pl.pallas_call(
        _body,
        out_shape=jax.ShapeDtypeStruct((B, SQ_PER, D), jnp.float32),
        in_specs=[pl.BlockSpec(memory_space=pltpu.VMEM)] * 7,
        out_specs=pl.BlockSpec(memory_space=pltpu.VMEM),
        scratch_shapes=[
            pltpu.VMEM((B, N_DEV, SQ_PER, D), jnp.bfloat16),
            pltpu.VMEM((ROWS, HD_PER), jnp.bfloat16),
            pltpu.VMEM((ROWS, HD_PER), jnp.bfloat16),
            pltpu.VMEM((ROWS, HD_PER), jnp.bfloat16),
            pltpu.VMEM((B, N_DEV, SQ_PER, D), jnp.bfloat16),
            pltpu.VMEM((B, SQ_PER, D), jnp.bfloat16),
            pltpu.SemaphoreType.DMA((N_DEV - 1,)),
            pltpu.SemaphoreType.DMA((N_DEV - 1,)),
            pltpu.SemaphoreType.DMA((N_DEV - 1,)),
            pltpu.SemaphoreType.DMA((N_DEV - 1,)),
            pltpu.SemaphoreType.REGULAR,
        ],
        compiler_params=pltpu.CompilerParams(
            collective_id=0,
            vmem_limit_bytes=100 * 1024 * 1024,
        ),
    )(x, Wq, Wk, Wv, Wo, cos, sin)
